# baseline (speedup 1.0000x reference)
import functools

import jax
import jax.numpy as jnp
from jax import lax
from jax.experimental import pallas as pl
from jax.experimental.pallas import tpu as pltpu
from jax.experimental.pallas import tpu_sc as plsc

N = 10000
H = 128
E = 320000
ATOM = 100
L = 3
NC, NS = 2, 16

EROWS = 2560
EPAD = EROWS * 128
ERPT = EROWS // (NC * NS)
NP = 10112
ZROWS = NP // NS
XROWS = 632
NXP = 12288
NXRT = NXP // 128 // (NC * NS)
NB = 1000


_SC_MESH = plsc.VectorSubcoreMesh(core_axis_name="c", subcore_axis_name="s",
                                  num_cores=NC, num_subcores=NS)


@functools.partial(
    pl.kernel,
    out_type=(
        jax.ShapeDtypeStruct((NXP, H), jnp.float32),
        jax.ShapeDtypeStruct((NC, N, H), jnp.float32),
    ),
    mesh=_SC_MESH,
    scratch_types=(
        pltpu.VMEM((16, 2, 128), jnp.int32),
        pltpu.VMEM((128,), jnp.int32),
        pltpu.VMEM((128, H), jnp.float32),
        pltpu.VMEM((128, H), jnp.float32),
        pltpu.VMEM_SHARED((ATOM, H), jnp.float32),
        pltpu.VMEM_SHARED((NP, H), jnp.float32),
        pltpu.SemaphoreType.DMA,
        pltpu.SemaphoreType.DMA,
        pltpu.SemaphoreType.DMA,
    ),
)
def _embed_deg(hpad, sd2, emb, zeros128, ones128,
               x, deg_out,
               sdall, idx, xbuf, onesbuf, emb_sp, deg_sp, sem, t0, t1):
    c = lax.axis_index("c")
    s = lax.axis_index("s")
    wid = c * NS + s

    @pl.when(s == 0)
    def _():
        pltpu.sync_copy(emb, emb_sp)

    pltpu.sync_copy(zeros128, deg_sp.at[pl.ds(s * ZROWS, ZROWS)])
    pltpu.sync_copy(ones128, onesbuf)
    plsc.subcore_barrier()

    for j in range(NXRT):
        r = wid * NXRT + j
        pltpu.sync_copy(hpad.at[r], idx)
        pltpu.async_copy(emb_sp.at[idx], xbuf, sem).wait()
        pltpu.sync_copy(xbuf, x.at[pl.ds(r * 128, 128)])

    tsem = (t0, t1)

    def chunk(ch, carry):
        pltpu.sync_copy(sd2.at[pl.ds(wid * ERPT + ch * 16, 16)], sdall)

        def body(jo, carry2):
            for b in range(2):
                j = jo * 2 + b
                pltpu.async_copy(onesbuf, deg_sp.at[sdall.at[j, 1]],
                                 tsem[b], add=True)
            for b in range(2):
                j = jo * 2 + b
                pltpu.make_async_copy(onesbuf, deg_sp.at[sdall.at[j, 1]],
                                      tsem[b]).wait()
            return carry2

        lax.fori_loop(0, 8, body, 0)
        return carry

    lax.fori_loop(0, ERPT // 16, chunk, 0)
    plsc.subcore_barrier()
    base = jnp.minimum(s * XROWS, N - XROWS)
    pltpu.sync_copy(deg_sp.at[pl.ds(base, XROWS)],
                    deg_out.at[c, pl.ds(base, XROWS)])


NBUF = 2
CH = 40
NCHK = ERPT // CH

CF = 0
RF_T = 120
RS_T = 40
RFTOT = RF_T * NS
EROWS_A = EROWS + CH


@functools.partial(
    pl.kernel,
    out_type=jax.ShapeDtypeStruct((NC, N, H), jnp.float32),
    mesh=_SC_MESH,
    scratch_types=(
        pltpu.VMEM((CH, 2, 128), jnp.int32),
        pltpu.VMEM((NBUF, 128, H), jnp.float32),
        pltpu.VMEM_SHARED((NP, H), jnp.float32),
        pltpu.SemaphoreType.DMA,
        pltpu.SemaphoreType.DMA,
        pltpu.SemaphoreType.DMA,
        pltpu.SemaphoreType.DMA,
    ),
)
def _agg(hfull, sd2, zeros128, aggp, sdall, msg, agg_sp, g0, g1, s0, s1):
    gsem = (g0, g1)
    ssem = (s0, s1)
    c = lax.axis_index("c")
    s = lax.axis_index("s")
    wid = c * NS + s

    pltpu.sync_copy(zeros128, agg_sp.at[pl.ds(s * ZROWS, ZROWS)])
    plsc.subcore_barrier()

    myrows = jnp.where(c == CF, RF_T, RS_T)
    mybase = jnp.where(c == CF, s * RF_T, RFTOT + s * RS_T)
    nchunks = (myrows + CH - 1) // CH

    def chunk(ch, carry):
        pltpu.sync_copy(sd2.at[pl.ds(mybase + ch * CH, CH)], sdall)
        rem = myrows - ch * CH
        lim = jnp.minimum(rem, CH)
        for b in range(NBUF):
            @pl.when(b < rem)
            def _():
                pltpu.async_copy(hfull.at[sdall.at[b, 0]], msg.at[b], gsem[b])

        def body(jo, carry2):
            for b in range(NBUF):
                j = jo * NBUF + b

                @pl.when(j < rem)
                def _():
                    pltpu.make_async_copy(hfull.at[sdall.at[j, 0]],
                                          msg.at[b], gsem[b]).wait()
                    pltpu.async_copy(msg.at[b], agg_sp.at[sdall.at[j, 1]],
                                     ssem[b], add=True)
                    pltpu.make_async_copy(msg.at[b], agg_sp.at[sdall.at[j, 1]],
                                          ssem[b]).wait()
                    jn = j + NBUF

                    @pl.when(jn < lim)
                    def _():
                        pltpu.async_copy(hfull.at[sdall.at[jn, 0]],
                                         msg.at[b], gsem[b])

            return carry2

        lax.fori_loop(0, CH // NBUF, body, 0)
        return carry

    lax.fori_loop(0, nchunks, chunk, 0)
    plsc.subcore_barrier()

    base = jnp.minimum(s * XROWS, N - XROWS)
    pltpu.sync_copy(agg_sp.at[pl.ds(base, XROWS)],
                    aggp.at[c, pl.ds(base, XROWS)])


def _layer_body(with_csi, hr, ar, dgr, wsr, wnr, br,
                outr, csr, *maybe_csir):
    i = pl.program_id(0)
    hc = hr[...]
    ag = ar[...]
    agg = ag[0] + ag[1]
    dg = dgr[...]
    inv = 1.0 / jnp.maximum(dg[0, :, 0:1] + dg[1, :, 0:1], 1.0)
    out = jnp.dot(hc, wsr[...], preferred_element_type=jnp.float32)
    out += jnp.dot(agg * inv, wnr[...], preferred_element_type=jnp.float32)
    out += br[...][0:1, :]
    out = jnp.maximum(out, 0.0)
    outr[...] = out

    @pl.when(i == 0)
    def _():
        csr[...] = jnp.zeros_like(csr)

    csr[...] += jnp.sum(out.reshape(NB // 8, 8, H), axis=0)
    if with_csi:
        (csir,) = maybe_csir

        @pl.when(i == 0)
        def _():
            csir[...] = jnp.zeros_like(csir)

        csir[...] += jnp.sum(hc.reshape(NB // 8, 8, H), axis=0)


def _layer_tc(hfull, aggp, deg16, ws, wn, b8, with_csi):
    blk = lambda i: (i, 0)
    fixed = lambda i: (0, 0)
    out_shape = [
        jax.ShapeDtypeStruct((NXP, H), jnp.float32),
        jax.ShapeDtypeStruct((8, H), jnp.float32),
    ]
    out_specs = [
        pl.BlockSpec((NB, H), blk),
        pl.BlockSpec((8, H), fixed),
    ]
    if with_csi:
        out_shape.append(jax.ShapeDtypeStruct((8, H), jnp.float32))
        out_specs.append(pl.BlockSpec((8, H), fixed))
    return pl.pallas_call(
        functools.partial(_layer_body, with_csi),
        grid=(N // NB,),
        in_specs=[
            pl.BlockSpec((NB, H), blk),
            pl.BlockSpec((NC, NB, H), lambda i: (0, i, 0)),
            pl.BlockSpec((NC, NB, H), lambda i: (0, i, 0)),
            pl.BlockSpec((H, H), fixed),
            pl.BlockSpec((H, H), fixed),
            pl.BlockSpec((8, H), fixed),
        ],
        out_specs=out_specs,
        out_shape=out_shape,
    )(hfull, aggp, deg16, ws, wn, b8)


def _final_body(csxr, cs1r, cs2r, cs3r, w1r, b1r, w2r, b2r, w3r, b3r, outr):
    hg = jnp.concatenate(
        [jnp.sum(csxr[...], axis=0, keepdims=True),
         jnp.sum(cs1r[...], axis=0, keepdims=True),
         jnp.sum(cs2r[...], axis=0, keepdims=True),
         jnp.sum(cs3r[...], axis=0, keepdims=True)], axis=1) * (1.0 / N)
    o = jnp.dot(hg, w1r[...], preferred_element_type=jnp.float32)
    o = jnp.maximum(o + b1r[...][0:1, :], 0.0)
    o = jnp.dot(o, w2r[...], preferred_element_type=jnp.float32)
    o = jnp.maximum(o + b2r[...][0:1, :], 0.0)
    o = jnp.dot(o, w3r[...], preferred_element_type=jnp.float32)
    o = o + b3r[...][0:1, :]
    outr[...] = jnp.broadcast_to(o, (8, H))


def _final_tc(csx, cs1, cs2, cs3, w1, b1, w2, b2, w3, b3):
    return pl.pallas_call(
        _final_body,
        out_shape=jax.ShapeDtypeStruct((8, H), jnp.float32),
    )(csx, cs1, cs2, cs3, w1, b1, w2, b2, w3, b3)


def kernel(h, edge_index, e, embed_table, W_self, W_neigh, b_layers,
           Wr1, br1, Wr2, br2, Wr3, br3):
    f32 = jnp.float32
    h = h.astype(jnp.int32)
    src = edge_index[0].astype(jnp.int32)
    dst = edge_index[1].astype(jnp.int32)
    npad = EROWS_A * 128 - E
    pad_dst = N + jnp.arange(npad, dtype=jnp.int32) % (NP - N)
    src_p = jnp.concatenate([src, jnp.zeros((npad,), jnp.int32)]).reshape(EROWS_A, 128)
    dst_p = jnp.concatenate([dst, pad_dst]).reshape(EROWS_A, 128)
    sd2 = jnp.stack([src_p, dst_p], axis=1)
    hpad = jnp.concatenate([h, jnp.zeros((NXP - N,), jnp.int32)]).reshape(NXP // 128, 128)
    zeros128 = jnp.zeros((ZROWS, H), f32)
    ones128 = jnp.ones((128, H), f32)

    x, deg16 = _embed_deg(hpad, sd2, embed_table, zeros128, ones128)

    hcur = x
    csx = None
    colsums = []
    for i in range(L):
        aggp = _agg(hcur, sd2, zeros128)
        b8 = jnp.broadcast_to(b_layers[i][None, :], (8, H))
        if i == 0:
            hcur, cs, csx = _layer_tc(hcur, aggp, deg16,
                                      W_self[i], W_neigh[i], b8, True)
        else:
            hcur, cs = _layer_tc(hcur, aggp, deg16,
                                 W_self[i], W_neigh[i], b8, False)
        colsums.append(cs)

    w3p = jnp.pad(Wr3, ((0, 0), (0, H - 1)))
    b1p = jnp.broadcast_to(br1[None, :], (8, 2 * H))
    b2p = jnp.broadcast_to(br2[None, :], (8, H))
    b3p = jnp.broadcast_to(jnp.pad(br3, (0, H - 1))[None, :], (8, H))
    outp = _final_tc(csx, colsums[0], colsums[1], colsums[2],
                     Wr1, b1p, Wr2, b2p, w3p, b3p)
    return outp[0:1, 0:1]


# speedup vs baseline: 1.1367x; 1.1367x over previous
import functools

import jax
import jax.numpy as jnp
from jax import lax
from jax.experimental import pallas as pl
from jax.experimental.pallas import tpu as pltpu
from jax.experimental.pallas import tpu_sc as plsc

N = 10000
H = 128
E = 320000
ATOM = 100
L = 3
NC, NS = 2, 16

EROWS = 2560
EPAD = EROWS * 128
ERPT = EROWS // (NC * NS)
NP = 10112
ZROWS = NP // NS
XROWS = 632
NXP = 12288
NXRT = NXP // 128 // (NC * NS)
NB = 1000


_SC_MESH = plsc.VectorSubcoreMesh(core_axis_name="c", subcore_axis_name="s",
                                  num_cores=NC, num_subcores=NS)


@functools.partial(
    pl.kernel,
    out_type=(
        jax.ShapeDtypeStruct((NXP, H), jnp.float32),
        jax.ShapeDtypeStruct((NC, N, H), jnp.float32),
    ),
    mesh=_SC_MESH,
    scratch_types=(
        pltpu.VMEM((16, 2, 128), jnp.int32),
        pltpu.VMEM((128,), jnp.int32),
        pltpu.VMEM((128, H), jnp.float32),
        pltpu.VMEM((128, H), jnp.float32),
        pltpu.VMEM_SHARED((ATOM, H), jnp.float32),
        pltpu.VMEM_SHARED((NP, H), jnp.float32),
        pltpu.SemaphoreType.DMA,
        pltpu.SemaphoreType.DMA,
        pltpu.SemaphoreType.DMA,
    ),
)
def _embed_deg(hpad, sd2, emb, zeros128, ones128,
               x, deg_out,
               sdall, idx, xbuf, onesbuf, emb_sp, deg_sp, sem, t0, t1):
    c = lax.axis_index("c")
    s = lax.axis_index("s")
    wid = c * NS + s

    @pl.when(s == 0)
    def _():
        pltpu.sync_copy(emb, emb_sp)

    pltpu.sync_copy(zeros128, deg_sp.at[pl.ds(s * ZROWS, ZROWS)])
    pltpu.sync_copy(ones128, onesbuf)
    plsc.subcore_barrier()

    for j in range(NXRT):
        r = wid * NXRT + j
        pltpu.sync_copy(hpad.at[r], idx)
        pltpu.async_copy(emb_sp.at[idx], xbuf, sem).wait()
        pltpu.sync_copy(xbuf, x.at[pl.ds(r * 128, 128)])

    tsem = (t0, t1)

    def chunk(ch, carry):
        pltpu.sync_copy(sd2.at[pl.ds(wid * ERPT + ch * 16, 16)], sdall)

        def body(jo, carry2):
            for b in range(2):
                j = jo * 2 + b
                pltpu.async_copy(onesbuf, deg_sp.at[sdall.at[j, 1]],
                                 tsem[b], add=True)
            for b in range(2):
                j = jo * 2 + b
                pltpu.make_async_copy(onesbuf, deg_sp.at[sdall.at[j, 1]],
                                      tsem[b]).wait()
            return carry2

        lax.fori_loop(0, 8, body, 0)
        return carry

    lax.fori_loop(0, ERPT // 16, chunk, 0)
    plsc.subcore_barrier()
    base = jnp.minimum(s * XROWS, N - XROWS)
    pltpu.sync_copy(deg_sp.at[pl.ds(base, XROWS)],
                    deg_out.at[c, pl.ds(base, XROWS)])


NBUF = 2
CH = 40
NCHK = ERPT // CH

CF = 0
RF_T = 150
RS_T = 10
RFTOT = RF_T * NS
EROWS_A = EROWS + CH


@functools.partial(
    pl.kernel,
    out_type=jax.ShapeDtypeStruct((NC, N, H), jnp.float32),
    mesh=_SC_MESH,
    scratch_types=(
        pltpu.VMEM((CH, 2, 128), jnp.int32),
        pltpu.VMEM((NBUF, 128, H), jnp.float32),
        pltpu.VMEM_SHARED((NP, H), jnp.float32),
        pltpu.SemaphoreType.DMA,
        pltpu.SemaphoreType.DMA,
        pltpu.SemaphoreType.DMA,
        pltpu.SemaphoreType.DMA,
    ),
)
def _agg(hfull, sd2, zeros128, aggp, sdall, msg, agg_sp, g0, g1, s0, s1):
    gsem = (g0, g1)
    ssem = (s0, s1)
    c = lax.axis_index("c")
    s = lax.axis_index("s")
    wid = c * NS + s

    pltpu.sync_copy(zeros128, agg_sp.at[pl.ds(s * ZROWS, ZROWS)])
    plsc.subcore_barrier()

    myrows = jnp.where(c == CF, RF_T, RS_T)
    mybase = jnp.where(c == CF, s * RF_T, RFTOT + s * RS_T)
    nchunks = (myrows + CH - 1) // CH

    def chunk(ch, carry):
        pltpu.sync_copy(sd2.at[pl.ds(mybase + ch * CH, CH)], sdall)
        rem = myrows - ch * CH
        lim = jnp.minimum(rem, CH)
        for b in range(NBUF):
            @pl.when(b < rem)
            def _():
                pltpu.async_copy(hfull.at[sdall.at[b, 0]], msg.at[b], gsem[b])

        def body(jo, carry2):
            for b in range(NBUF):
                j = jo * NBUF + b

                @pl.when(j < rem)
                def _():
                    pltpu.make_async_copy(hfull.at[sdall.at[j, 0]],
                                          msg.at[b], gsem[b]).wait()
                    pltpu.async_copy(msg.at[b], agg_sp.at[sdall.at[j, 1]],
                                     ssem[b], add=True)
                    pltpu.make_async_copy(msg.at[b], agg_sp.at[sdall.at[j, 1]],
                                          ssem[b]).wait()
                    jn = j + NBUF

                    @pl.when(jn < lim)
                    def _():
                        pltpu.async_copy(hfull.at[sdall.at[jn, 0]],
                                         msg.at[b], gsem[b])

            return carry2

        lax.fori_loop(0, CH // NBUF, body, 0)
        return carry

    lax.fori_loop(0, nchunks, chunk, 0)
    plsc.subcore_barrier()

    base = jnp.minimum(s * XROWS, N - XROWS)
    pltpu.sync_copy(agg_sp.at[pl.ds(base, XROWS)],
                    aggp.at[c, pl.ds(base, XROWS)])


def _layer_body(with_csi, hr, ar, dgr, wsr, wnr, br,
                outr, csr, *maybe_csir):
    i = pl.program_id(0)
    hc = hr[...]
    ag = ar[...]
    agg = ag[0] + ag[1]
    dg = dgr[...]
    inv = 1.0 / jnp.maximum(dg[0, :, 0:1] + dg[1, :, 0:1], 1.0)
    out = jnp.dot(hc, wsr[...], preferred_element_type=jnp.float32)
    out += jnp.dot(agg * inv, wnr[...], preferred_element_type=jnp.float32)
    out += br[...][0:1, :]
    out = jnp.maximum(out, 0.0)
    outr[...] = out

    @pl.when(i == 0)
    def _():
        csr[...] = jnp.zeros_like(csr)

    csr[...] += jnp.sum(out.reshape(NB // 8, 8, H), axis=0)
    if with_csi:
        (csir,) = maybe_csir

        @pl.when(i == 0)
        def _():
            csir[...] = jnp.zeros_like(csir)

        csir[...] += jnp.sum(hc.reshape(NB // 8, 8, H), axis=0)


def _layer_tc(hfull, aggp, deg16, ws, wn, b8, with_csi):
    blk = lambda i: (i, 0)
    fixed = lambda i: (0, 0)
    out_shape = [
        jax.ShapeDtypeStruct((NXP, H), jnp.float32),
        jax.ShapeDtypeStruct((8, H), jnp.float32),
    ]
    out_specs = [
        pl.BlockSpec((NB, H), blk),
        pl.BlockSpec((8, H), fixed),
    ]
    if with_csi:
        out_shape.append(jax.ShapeDtypeStruct((8, H), jnp.float32))
        out_specs.append(pl.BlockSpec((8, H), fixed))
    return pl.pallas_call(
        functools.partial(_layer_body, with_csi),
        grid=(N // NB,),
        in_specs=[
            pl.BlockSpec((NB, H), blk),
            pl.BlockSpec((NC, NB, H), lambda i: (0, i, 0)),
            pl.BlockSpec((NC, NB, H), lambda i: (0, i, 0)),
            pl.BlockSpec((H, H), fixed),
            pl.BlockSpec((H, H), fixed),
            pl.BlockSpec((8, H), fixed),
        ],
        out_specs=out_specs,
        out_shape=out_shape,
    )(hfull, aggp, deg16, ws, wn, b8)


def _final_body(csxr, cs1r, cs2r, cs3r, w1r, b1r, w2r, b2r, w3r, b3r, outr):
    hg = jnp.concatenate(
        [jnp.sum(csxr[...], axis=0, keepdims=True),
         jnp.sum(cs1r[...], axis=0, keepdims=True),
         jnp.sum(cs2r[...], axis=0, keepdims=True),
         jnp.sum(cs3r[...], axis=0, keepdims=True)], axis=1) * (1.0 / N)
    o = jnp.dot(hg, w1r[...], preferred_element_type=jnp.float32)
    o = jnp.maximum(o + b1r[...][0:1, :], 0.0)
    o = jnp.dot(o, w2r[...], preferred_element_type=jnp.float32)
    o = jnp.maximum(o + b2r[...][0:1, :], 0.0)
    o = jnp.dot(o, w3r[...], preferred_element_type=jnp.float32)
    o = o + b3r[...][0:1, :]
    outr[...] = jnp.broadcast_to(o, (8, H))


def _final_tc(csx, cs1, cs2, cs3, w1, b1, w2, b2, w3, b3):
    return pl.pallas_call(
        _final_body,
        out_shape=jax.ShapeDtypeStruct((8, H), jnp.float32),
    )(csx, cs1, cs2, cs3, w1, b1, w2, b2, w3, b3)


def kernel(h, edge_index, e, embed_table, W_self, W_neigh, b_layers,
           Wr1, br1, Wr2, br2, Wr3, br3):
    f32 = jnp.float32
    h = h.astype(jnp.int32)
    src = edge_index[0].astype(jnp.int32)
    dst = edge_index[1].astype(jnp.int32)
    npad = EROWS_A * 128 - E
    pad_dst = N + jnp.arange(npad, dtype=jnp.int32) % (NP - N)
    src_p = jnp.concatenate([src, jnp.zeros((npad,), jnp.int32)]).reshape(EROWS_A, 128)
    dst_p = jnp.concatenate([dst, pad_dst]).reshape(EROWS_A, 128)
    sd2 = jnp.stack([src_p, dst_p], axis=1)
    hpad = jnp.concatenate([h, jnp.zeros((NXP - N,), jnp.int32)]).reshape(NXP // 128, 128)
    zeros128 = jnp.zeros((ZROWS, H), f32)
    ones128 = jnp.ones((128, H), f32)

    x, deg16 = _embed_deg(hpad, sd2, embed_table, zeros128, ones128)

    hcur = x
    csx = None
    colsums = []
    for i in range(L):
        aggp = _agg(hcur, sd2, zeros128)
        b8 = jnp.broadcast_to(b_layers[i][None, :], (8, H))
        if i == 0:
            hcur, cs, csx = _layer_tc(hcur, aggp, deg16,
                                      W_self[i], W_neigh[i], b8, True)
        else:
            hcur, cs = _layer_tc(hcur, aggp, deg16,
                                 W_self[i], W_neigh[i], b8, False)
        colsums.append(cs)

    w3p = jnp.pad(Wr3, ((0, 0), (0, H - 1)))
    b1p = jnp.broadcast_to(br1[None, :], (8, 2 * H))
    b2p = jnp.broadcast_to(br2[None, :], (8, H))
    b3p = jnp.broadcast_to(jnp.pad(br3, (0, H - 1))[None, :], (8, H))
    outp = _final_tc(csx, colsums[0], colsums[1], colsums[2],
                     Wr1, b1p, Wr2, b2p, w3p, b3p)
    return outp[0:1, 0:1]
